# Initial kernel scaffold; baseline (speedup 1.0000x reference)
import jax
import jax.numpy as jnp
from jax.experimental import pallas as pl


def kernel(x, edge_index, W1, b1, W2, b2):
    raise NotImplementedError("write your pallas kernel here")


# speedup vs baseline: 23.2252x; 23.2252x over previous
import functools

import jax
import jax.numpy as jnp
from jax import lax
from jax.experimental import pallas as pl
from jax.experimental.pallas import tpu as pltpu
from jax.experimental.pallas import tpu_sc as plsc

N = 10000
NP = 10112
E = 320000
D = 128

NC = 2
NS = 16
NW = NC * NS
EPW = E // NW
K = 80
CH = EPW // K
ROWS_PT = NP // NS


@functools.cache
def _get_agg_kernel():
    mesh = plsc.VectorSubcoreMesh(
        core_axis_name="c", subcore_axis_name="s", num_cores=NC, num_subcores=NS
    )
    return pl.kernel(
        _agg_body,
        out_type=jax.ShapeDtypeStruct((NC, NP, D), jnp.float32),
        mesh=mesh,
        scratch_types=[
            pltpu.VMEM((EPW,), jnp.int32),
            pltpu.VMEM((EPW,), jnp.int32),
            pltpu.VMEM((K, D), jnp.float32),
            pltpu.VMEM((K, D), jnp.float32),
            pltpu.SemaphoreType.DMA,
            pltpu.SemaphoreType.DMA,
            pltpu.VMEM_SHARED((NP, D), jnp.float32),
        ],
    )


def _agg_body(g_hbm, src_hbm, dst_hbm, out_hbm,
              src_v, dst_v, rows0, rows1, sem0, sem1, acc_sh):
    cid = lax.axis_index("c")
    sid = lax.axis_index("s")
    wid = cid * NS + sid
    rows = (rows0, rows1)
    sems = (sem0, sem1)

    sl = pl.ds(sid * ROWS_PT, ROWS_PT)
    pltpu.sync_copy(g_hbm.at[sl], acc_sh.at[sl])
    pltpu.sync_copy(src_hbm.at[wid], src_v)
    pltpu.sync_copy(dst_hbm.at[wid], dst_v)
    plsc.subcore_barrier()

    def sidx(j):
        return src_v.at[pl.ds(j * K, K)]

    def didx(j):
        return dst_v.at[pl.ds(j * K, K)]

    pltpu.async_copy(g_hbm.at[sidx(0)], rows[0], sems[0])
    pltpu.async_copy(g_hbm.at[sidx(1)], rows[1], sems[1])

    def step(j, b):
        pltpu.make_async_copy(g_hbm.at[sidx(j)], rows[b], sems[b]).wait()
        pltpu.sync_copy(rows[b], acc_sh.at[didx(j)], add=True)

        @pl.when(j + 2 < CH)
        def _():
            pltpu.async_copy(g_hbm.at[sidx(j + 2)], rows[b], sems[b])

    def body(i, _):
        step(2 * i, 0)
        step(2 * i + 1, 1)
        return ()

    lax.fori_loop(0, (CH - 1) // 2, body, (), unroll=False)
    step(CH - 1, 0)

    plsc.subcore_barrier()
    pltpu.sync_copy(acc_sh.at[sl], out_hbm.at[cid, sl])


R = 1264


def _scale1_body(degp_ref, x_ref, w_ref, g_ref, dinv_ref):
    deg = degp_ref[0, :, 0:1] + degp_ref[1, :, 0:1] - 1.0
    dinv = lax.rsqrt(deg)
    h = jnp.dot(x_ref[...], w_ref[...], preferred_element_type=jnp.float32)
    g_ref[...] = dinv * h
    dinv_ref[...] = jnp.broadcast_to(dinv, (R, D))


def _mm_scale1(degp, x, w1):
    return pl.pallas_call(
        _scale1_body,
        grid=(NP // R,),
        in_specs=[
            pl.BlockSpec((NC, R, D), lambda i: (0, i, 0)),
            pl.BlockSpec((R, D), lambda i: (i, 0)),
            pl.BlockSpec((D, D), lambda i: (0, 0)),
        ],
        out_specs=[
            pl.BlockSpec((R, D), lambda i: (i, 0)),
            pl.BlockSpec((R, D), lambda i: (i, 0)),
        ],
        out_shape=[
            jax.ShapeDtypeStruct((NP, D), jnp.float32),
            jax.ShapeDtypeStruct((NP, D), jnp.float32),
        ],
    )(degp, x, w1)


def _mid_body(acc_ref, g1_ref, dinv_ref, w_ref, b1_ref, g2_ref):
    dinv = dinv_ref[...]
    agg = acc_ref[0] + acc_ref[1] - g1_ref[...]
    h = jnp.maximum(dinv * agg + b1_ref[...], 0.0)
    g2_ref[...] = dinv * jnp.dot(h, w_ref[...], preferred_element_type=jnp.float32)


def _mid(acc1, g1, dinvb, w2, b1):
    return pl.pallas_call(
        _mid_body,
        grid=(NP // R,),
        in_specs=[
            pl.BlockSpec((NC, R, D), lambda i: (0, i, 0)),
            pl.BlockSpec((R, D), lambda i: (i, 0)),
            pl.BlockSpec((R, D), lambda i: (i, 0)),
            pl.BlockSpec((D, D), lambda i: (0, 0)),
            pl.BlockSpec((1, D), lambda i: (0, 0)),
        ],
        out_specs=pl.BlockSpec((R, D), lambda i: (i, 0)),
        out_shape=jax.ShapeDtypeStruct((NP, D), jnp.float32),
    )(acc1, g1, dinvb, w2, b1)


def _final_body(acc_ref, g2_ref, dinv_ref, b2_ref, out_ref):
    agg = acc_ref[0] + acc_ref[1] - g2_ref[...]
    z = dinv_ref[...] * agg + b2_ref[...]
    m = jnp.max(z, axis=1, keepdims=True)
    lse = jnp.log(jnp.sum(jnp.exp(z - m), axis=1, keepdims=True))
    out_ref[...] = z - m - lse


def _final(acc2, g2, dinvb, b2):
    return pl.pallas_call(
        _final_body,
        grid=(NP // R,),
        in_specs=[
            pl.BlockSpec((NC, R, D), lambda i: (0, i, 0)),
            pl.BlockSpec((R, D), lambda i: (i, 0)),
            pl.BlockSpec((R, D), lambda i: (i, 0)),
            pl.BlockSpec((1, D), lambda i: (0, 0)),
        ],
        out_specs=pl.BlockSpec((R, D), lambda i: (i, 0)),
        out_shape=jax.ShapeDtypeStruct((NP, D), jnp.float32),
    )(acc2, g2, dinvb, b2)


def kernel(x, edge_index, W1, b1, W2, b2):
    src3 = edge_index[0].reshape(NW, EPW)
    dst3 = edge_index[1].reshape(NW, EPW)
    xp = jnp.pad(x, ((0, NP - N), (0, 0)))
    b1r = b1.reshape(1, D)
    b2r = b2.reshape(1, D)

    degp = _get_agg_kernel()(jnp.ones((NP, D), jnp.float32), src3, dst3)
    g1, dinvb = _mm_scale1(degp, xp, W1)
    acc1 = _get_agg_kernel()(g1, src3, dst3)
    g2 = _mid(acc1, g1, dinvb, W2, b1r)
    acc2 = _get_agg_kernel()(g2, src3, dst3)
    return _final(acc2, g2, dinvb, b2r)[:N]
